# baseline (speedup 1.0000x reference)
import functools

import jax
import jax.numpy as jnp
from jax.experimental import pallas as pl
from jax.experimental.pallas import tpu as pltpu


def _proj_body(x_ref, w_ref, b_ref, z_ref, idx_ref, o_ref, *, bm):
    i = pl.program_id(1)
    xb = x_ref[0]
    h = jnp.dot(xb.astype(jnp.bfloat16), w_ref[...].astype(jnp.bfloat16),
                preferred_element_type=jnp.float32) + b_ref[...]
    rows = jax.lax.broadcasted_iota(jnp.int32, (bm, 1), 0)
    local = idx_ref[0, 0, 0] - i * bm
    o_ref[0] = jnp.where(rows == local, z_ref[0], h)


def _proj(x, W, bias, z3, idx3, bm):
    b, n, din = x.shape
    dh = W.shape[1]
    return pl.pallas_call(
        functools.partial(_proj_body, bm=bm),
        grid=(b, n // bm),
        in_specs=[
            pl.BlockSpec((1, bm, din), lambda bb, ii: (bb, ii, 0)),
            pl.BlockSpec((din, dh), lambda bb, ii: (0, 0)),
            pl.BlockSpec((1, dh), lambda bb, ii: (0, 0)),
            pl.BlockSpec((1, 1, dh), lambda bb, ii: (bb, 0, 0)),
            pl.BlockSpec((1, 1, 1), lambda bb, ii: (bb, 0, 0)),
        ],
        out_specs=pl.BlockSpec((1, bm, dh), lambda bb, ii: (bb, ii, 0)),
        out_shape=jax.ShapeDtypeStruct((b, n, dh), jnp.float32),
        compiler_params=pltpu.CompilerParams(
            dimension_semantics=("parallel", "parallel")),
    )(x, W, bias, z3, idx3)


def _scores(hz, w_ref, asrc_ref, adst_ref):
    g = jnp.dot(hz.astype(jnp.bfloat16), w_ref[...].astype(jnp.bfloat16),
                preferred_element_type=jnp.float32)
    ssrc = jax.lax.dot_general(g, asrc_ref[...], (((1,), (1,)), ((), ())),
                               preferred_element_type=jnp.float32)
    sdst = jax.lax.dot_general(adst_ref[...], g, (((1,), (1,)), ((), ())),
                               preferred_element_type=jnp.float32)
    return g, ssrc, sdst


def _attn_rows(v, lut_ref, ssrc_blk, sdst, g_bf, nt, tb_ref, bm, ntyp):
    tab = jnp.broadcast_to(lut_ref[...], (bm, lut_ref.shape[1]))
    lut = jnp.take_along_axis(tab, v, axis=1)
    e0 = ssrc_blk + sdst + lut
    e = jnp.maximum(e0, 0.2 * e0)
    m = jnp.max(e, axis=1, keepdims=True)
    p = jnp.exp2(e - m)
    l = jnp.sum(p, axis=1, keepdims=True)
    pm = jnp.where(v > 0, p, 0.0)
    acc = jnp.dot(pm.astype(jnp.bfloat16), g_bf,
                  preferred_element_type=jnp.float32)
    oneh = (nt == jax.lax.broadcasted_iota(jnp.int32, (bm, ntyp), 1)
            ).astype(jnp.float32)
    tbias = jnp.dot(oneh, tb_ref[...], preferred_element_type=jnp.float32)
    return jnp.maximum(acc * (1.0 / l) + tbias, 0.0)


def _accum_rows(i, idx_ref, hout, zrow_ref, rsum_ref, bm):
    rows_g = jax.lax.broadcasted_iota(jnp.int32, (bm, 1), 0) + i * bm
    zc = jnp.sum(jnp.where(rows_g == idx_ref[0, 0, 0], hout, 0.0),
                 axis=0, keepdims=True)
    rs = jnp.sum(hout, axis=0, keepdims=True)

    @pl.when(i == 0)
    def _():
        zrow_ref[0] = zc
        rsum_ref[0] = rs

    @pl.when(i > 0)
    def _():
        zrow_ref[0] += zc
        rsum_ref[0] += rs


def _layer1_body(h_ref, w_ref, asrc_ref, adst_ref, z_ref, idx_ref,
                 adj_ref, et_ref, nt_ref, lut_ref, tb_ref,
                 ho_ref, v_ref, zrow_ref, rsum_ref,
                 gb_scr, ssrc_scr, sdst_scr, *, bm, ntyp):
    i = pl.program_id(1)

    @pl.when(i == 0)
    def _():
        h = h_ref[0]
        n = h.shape[0]
        rows = jax.lax.broadcasted_iota(jnp.int32, (n, 1), 0)
        hz = jnp.where(rows == idx_ref[0, 0, 0], z_ref[0], h)
        g, ssrc, sdst = _scores(hz, w_ref, asrc_ref, adst_ref)
        gb_scr[...] = g.astype(jnp.bfloat16)
        ssrc_scr[...] = ssrc
        sdst_scr[...] = sdst

    et = et_ref[0]
    v = jnp.where(adj_ref[0] > 0.5, et + 1, 0)
    v_ref[0] = v.astype(jnp.int8)
    hout = _attn_rows(v, lut_ref, ssrc_scr[pl.ds(i * bm, bm), :],
                      sdst_scr[...], gb_scr[...], nt_ref[0], tb_ref, bm, ntyp)
    ho_ref[0] = hout
    _accum_rows(i, idx_ref, hout, zrow_ref, rsum_ref, bm)


def _layer1(h, W, a_src2, a_dst2, z3, idx3, adj, et, nt3, lut, tb, bm):
    b, n, dh = h.shape
    ntyp = tb.shape[0]
    return pl.pallas_call(
        functools.partial(_layer1_body, bm=bm, ntyp=ntyp),
        grid=(b, n // bm),
        in_specs=[
            pl.BlockSpec((1, n, dh), lambda bb, ii: (bb, 0, 0)),
            pl.BlockSpec((dh, dh), lambda bb, ii: (0, 0)),
            pl.BlockSpec((1, dh), lambda bb, ii: (0, 0)),
            pl.BlockSpec((1, dh), lambda bb, ii: (0, 0)),
            pl.BlockSpec((1, 1, dh), lambda bb, ii: (bb, 0, 0)),
            pl.BlockSpec((1, 1, 1), lambda bb, ii: (bb, 0, 0)),
            pl.BlockSpec((1, bm, n), lambda bb, ii: (bb, ii, 0)),
            pl.BlockSpec((1, bm, n), lambda bb, ii: (bb, ii, 0)),
            pl.BlockSpec((1, bm, 1), lambda bb, ii: (bb, ii, 0)),
            pl.BlockSpec((1, 128), lambda bb, ii: (0, 0)),
            pl.BlockSpec((ntyp, dh), lambda bb, ii: (0, 0)),
        ],
        out_specs=[
            pl.BlockSpec((1, bm, dh), lambda bb, ii: (bb, ii, 0)),
            pl.BlockSpec((1, bm, n), lambda bb, ii: (bb, ii, 0)),
            pl.BlockSpec((1, 1, dh), lambda bb, ii: (bb, 0, 0)),
            pl.BlockSpec((1, 1, dh), lambda bb, ii: (bb, 0, 0)),
        ],
        out_shape=[
            jax.ShapeDtypeStruct((b, n, dh), jnp.float32),
            jax.ShapeDtypeStruct((b, n, n), jnp.int8),
            jax.ShapeDtypeStruct((b, 1, dh), jnp.float32),
            jax.ShapeDtypeStruct((b, 1, dh), jnp.float32),
        ],
        scratch_shapes=[
            pltpu.VMEM((n, dh), jnp.bfloat16),
            pltpu.VMEM((n, 1), jnp.float32),
            pltpu.VMEM((1, n), jnp.float32),
        ],
        compiler_params=pltpu.CompilerParams(
            dimension_semantics=("parallel", "arbitrary")),
    )(h, W, a_src2, a_dst2, z3, idx3, adj, et, nt3, lut, tb)


def _layer2_body(h_ref, w_ref, asrc_ref, adst_ref, zrs_ref, zrc_ref,
                 fzw_ref, fzb_ref, idx_ref, v_ref, nt_ref, lut_ref, tb_ref,
                 zrow_ref, rsum_ref,
                 gb_scr, ssrc_scr, sdst_scr, *, bm, ntyp, dh):
    i = pl.program_id(1)

    @pl.when(i == 0)
    def _():
        fzw = fzw_ref[...]
        z = (jnp.dot(zrs_ref[0], fzw[:dh], preferred_element_type=jnp.float32)
             + jnp.dot(zrc_ref[0], fzw[dh:],
                       preferred_element_type=jnp.float32)
             + fzb_ref[...])
        h = h_ref[0]
        n = h.shape[0]
        rows = jax.lax.broadcasted_iota(jnp.int32, (n, 1), 0)
        hz = jnp.where(rows == idx_ref[0, 0, 0], z, h)
        g, ssrc, sdst = _scores(hz, w_ref, asrc_ref, adst_ref)
        gb_scr[...] = g.astype(jnp.bfloat16)
        ssrc_scr[...] = ssrc
        sdst_scr[...] = sdst

    v = v_ref[0].astype(jnp.int32)
    hout = _attn_rows(v, lut_ref, ssrc_scr[pl.ds(i * bm, bm), :],
                      sdst_scr[...], gb_scr[...], nt_ref[0], tb_ref, bm, ntyp)
    _accum_rows(i, idx_ref, hout, zrow_ref, rsum_ref, bm)


def _layer2(h, W, a_src2, a_dst2, zrow_s, zrow_c, f_z_W, f_z_b2, idx3,
            v, nt3, lut, tb, bm):
    b, n, dh = h.shape
    ntyp = tb.shape[0]
    return pl.pallas_call(
        functools.partial(_layer2_body, bm=bm, ntyp=ntyp, dh=dh),
        grid=(b, n // bm),
        in_specs=[
            pl.BlockSpec((1, n, dh), lambda bb, ii: (bb, 0, 0)),
            pl.BlockSpec((dh, dh), lambda bb, ii: (0, 0)),
            pl.BlockSpec((1, dh), lambda bb, ii: (0, 0)),
            pl.BlockSpec((1, dh), lambda bb, ii: (0, 0)),
            pl.BlockSpec((1, 1, dh), lambda bb, ii: (bb, 0, 0)),
            pl.BlockSpec((1, 1, dh), lambda bb, ii: (bb, 0, 0)),
            pl.BlockSpec((2 * dh, dh), lambda bb, ii: (0, 0)),
            pl.BlockSpec((1, dh), lambda bb, ii: (0, 0)),
            pl.BlockSpec((1, 1, 1), lambda bb, ii: (bb, 0, 0)),
            pl.BlockSpec((1, bm, n), lambda bb, ii: (bb, ii, 0)),
            pl.BlockSpec((1, bm, 1), lambda bb, ii: (bb, ii, 0)),
            pl.BlockSpec((1, 128), lambda bb, ii: (0, 0)),
            pl.BlockSpec((ntyp, dh), lambda bb, ii: (0, 0)),
        ],
        out_specs=[
            pl.BlockSpec((1, 1, dh), lambda bb, ii: (bb, 0, 0)),
            pl.BlockSpec((1, 1, dh), lambda bb, ii: (bb, 0, 0)),
        ],
        out_shape=[
            jax.ShapeDtypeStruct((b, 1, dh), jnp.float32),
            jax.ShapeDtypeStruct((b, 1, dh), jnp.float32),
        ],
        scratch_shapes=[
            pltpu.VMEM((n, dh), jnp.bfloat16),
            pltpu.VMEM((n, 1), jnp.float32),
            pltpu.VMEM((1, n), jnp.float32),
        ],
        compiler_params=pltpu.CompilerParams(
            dimension_semantics=("parallel", "arbitrary")),
    )(h, W, a_src2, a_dst2, zrow_s, zrow_c, f_z_W, f_z_b2, idx3, v, nt3,
      lut, tb)


def _head_body(zrs_ref, rss_ref, zrc_ref, rsc_ref, p_ref, fzw_ref, fzb_ref,
               hw_ref, hb_ref, o_ref, *, dh, n):
    zs = zrs_ref[:, 0, :]
    zc = zrc_ref[:, 0, :]
    pool_s = (rss_ref[:, 0, :] - zs) * (1.0 / (n - 1.0))
    pool_c = (rsc_ref[:, 0, :] - zc) * (1.0 / (n - 1.0))
    fzw = fzw_ref[...]
    z = (jnp.dot(zs, fzw[:dh], preferred_element_type=jnp.float32)
         + jnp.dot(zc, fzw[dh:], preferred_element_type=jnp.float32)
         + fzb_ref[...])
    hw = hw_ref[...]
    logit = (jnp.dot(pool_s, hw[:dh], preferred_element_type=jnp.float32)
             + jnp.dot(pool_c, hw[dh:2 * dh],
                       preferred_element_type=jnp.float32)
             + jnp.dot(p_ref[:, 0, :], hw[2 * dh:3 * dh],
                       preferred_element_type=jnp.float32)
             + jnp.dot(z, hw[3 * dh:], preferred_element_type=jnp.float32)
             + hb_ref[...])
    o_ref[...] = logit


def _head(zrow_s, rsum_s, zrow_c, rsum_c, p3, f_z_W, f_z_b2, head_W,
          head_b2, n):
    b, _, dh = zrow_s.shape
    return pl.pallas_call(
        functools.partial(_head_body, dh=dh, n=n),
        grid=(1,),
        in_specs=[
            pl.BlockSpec((b, 1, dh), lambda _: (0, 0, 0)),
            pl.BlockSpec((b, 1, dh), lambda _: (0, 0, 0)),
            pl.BlockSpec((b, 1, dh), lambda _: (0, 0, 0)),
            pl.BlockSpec((b, 1, dh), lambda _: (0, 0, 0)),
            pl.BlockSpec((b, 1, dh), lambda _: (0, 0, 0)),
            pl.BlockSpec((2 * dh, dh), lambda _: (0, 0)),
            pl.BlockSpec((1, dh), lambda _: (0, 0)),
            pl.BlockSpec((4 * dh, 1), lambda _: (0, 0)),
            pl.BlockSpec((1, 1), lambda _: (0, 0)),
        ],
        out_specs=pl.BlockSpec((b, 1), lambda _: (0, 0)),
        out_shape=jax.ShapeDtypeStruct((b, 1), jnp.float32),
    )(zrow_s, rsum_s, zrow_c, rsum_c, p3, f_z_W, f_z_b2, head_W, head_b2)


def _lut(rel_k):
    r = rel_k.shape[0]
    return jnp.concatenate(
        [jnp.full((1, 1), -1e9, jnp.float32), rel_k.reshape(1, r),
         jnp.zeros((1, 127 - r), jnp.float32)], axis=1)


def kernel(z_features, p_features, scene_node_features, scene_adj,
           scene_edge_types, scene_node_types, scene_node_is_z_index,
           concept_node_features, concept_adj, concept_edge_types,
           concept_node_types, concept_node_is_z_index,
           visual_W, visual_b, kg_W, kg_b, f_z_W, f_z_b, head_W, head_b,
           scene_W, scene_a_src, scene_a_dst, scene_rel, scene_tb,
           concept_W, concept_a_src, concept_a_dst, concept_rel, concept_tb):
    b, dh = z_features.shape
    ns = scene_adj.shape[1]
    nc = concept_adj.shape[1]
    bm = 1024

    idx_s3 = scene_node_is_z_index.astype(jnp.int32).reshape(b, 1, 1)
    idx_c3 = concept_node_is_z_index.astype(jnp.int32).reshape(b, 1, 1)
    nt_s3 = scene_node_types.astype(jnp.int32).reshape(b, ns, 1)
    nt_c3 = concept_node_types.astype(jnp.int32).reshape(b, nc, 1)
    et_s = scene_edge_types.astype(jnp.int32)
    et_c = concept_edge_types.astype(jnp.int32)
    z3 = z_features.reshape(b, 1, dh)
    p3 = p_features.reshape(b, 1, dh)
    vb2 = visual_b.reshape(1, dh)
    kb2 = kg_b.reshape(1, dh)
    fzb2 = f_z_b.reshape(1, dh)
    hb2 = head_b.reshape(1, 1)

    lg2 = jnp.float32(1.4426950408889634)

    h_s = _proj(scene_node_features, visual_W, vb2, z3, idx_s3, bm)
    h_c = _proj(concept_node_features, kg_W, kb2, z3, idx_c3, bm)

    h_s, v_s, zrow_s, _ = _layer1(
        h_s, scene_W[0], (scene_a_src[0] * lg2).reshape(1, dh),
        (scene_a_dst[0] * lg2).reshape(1, dh), z3, idx_s3, scene_adj, et_s,
        nt_s3, _lut(scene_rel[0] * lg2), scene_tb[0], bm)
    h_c, v_c, zrow_c, _ = _layer1(
        h_c, concept_W[0], (concept_a_src[0] * lg2).reshape(1, dh),
        (concept_a_dst[0] * lg2).reshape(1, dh), z3, idx_c3, concept_adj,
        et_c, nt_c3, _lut(concept_rel[0] * lg2), concept_tb[0], bm)

    zrow_s2, rsum_s2 = _layer2(
        h_s, scene_W[1], (scene_a_src[1] * lg2).reshape(1, dh),
        (scene_a_dst[1] * lg2).reshape(1, dh), zrow_s, zrow_c, f_z_W, fzb2,
        idx_s3, v_s, nt_s3, _lut(scene_rel[1] * lg2), scene_tb[1], bm)
    zrow_c2, rsum_c2 = _layer2(
        h_c, concept_W[1], (concept_a_src[1] * lg2).reshape(1, dh),
        (concept_a_dst[1] * lg2).reshape(1, dh), zrow_s, zrow_c, f_z_W, fzb2,
        idx_c3, v_c, nt_c3, _lut(concept_rel[1] * lg2), concept_tb[1], bm)

    return _head(zrow_s2, rsum_s2, zrow_c2, rsum_c2, p3, f_z_W, fzb2,
                 head_W, hb2, ns)


# speedup vs baseline: 1.0543x; 1.0543x over previous
import functools

import jax
import jax.numpy as jnp
from jax.experimental import pallas as pl
from jax.experimental.pallas import tpu as pltpu


def _proj_body(x_ref, w_ref, b_ref, z_ref, idx_ref, o_ref, *, bm):
    i = pl.program_id(1)
    xb = x_ref[0]
    h = jnp.dot(xb.astype(jnp.bfloat16), w_ref[...].astype(jnp.bfloat16),
                preferred_element_type=jnp.float32) + b_ref[...]
    rows = jax.lax.broadcasted_iota(jnp.int32, (bm, 1), 0)
    local = idx_ref[0, 0, 0] - i * bm
    o_ref[0] = jnp.where(rows == local, z_ref[0], h)


def _proj(x, W, bias, z3, idx3, bm):
    b, n, din = x.shape
    dh = W.shape[1]
    return pl.pallas_call(
        functools.partial(_proj_body, bm=bm),
        grid=(b, n // bm),
        in_specs=[
            pl.BlockSpec((1, bm, din), lambda bb, ii: (bb, ii, 0)),
            pl.BlockSpec((din, dh), lambda bb, ii: (0, 0)),
            pl.BlockSpec((1, dh), lambda bb, ii: (0, 0)),
            pl.BlockSpec((1, 1, dh), lambda bb, ii: (bb, 0, 0)),
            pl.BlockSpec((1, 1, 1), lambda bb, ii: (bb, 0, 0)),
        ],
        out_specs=pl.BlockSpec((1, bm, dh), lambda bb, ii: (bb, ii, 0)),
        out_shape=jax.ShapeDtypeStruct((b, n, dh), jnp.float32),
        compiler_params=pltpu.CompilerParams(
            dimension_semantics=("parallel", "parallel")),
    )(x, W, bias, z3, idx3)


def _scores(hz, w_ref, asrc_ref, adst_ref):
    g = jnp.dot(hz.astype(jnp.bfloat16), w_ref[...].astype(jnp.bfloat16),
                preferred_element_type=jnp.float32)
    ssrc = jax.lax.dot_general(g, asrc_ref[...], (((1,), (1,)), ((), ())),
                               preferred_element_type=jnp.float32)
    sdst = jax.lax.dot_general(adst_ref[...], g, (((1,), (1,)), ((), ())),
                               preferred_element_type=jnp.float32)
    return g, ssrc, sdst


def _attn_rows(v, lut_ref, ssrc_blk, sdst, g_bf, nt, tb_ref, bm, ntyp, dh):
    tab = jnp.broadcast_to(lut_ref[...], (bm, lut_ref.shape[1]))
    lut = jnp.take_along_axis(tab, v, axis=1)
    e0 = ssrc_blk + sdst + lut
    e = jnp.maximum(e0, 0.2 * e0)
    m = jnp.max(e, axis=1, keepdims=True)
    p = jnp.exp2(e - m)
    pm = jnp.where(v > 0, p, 0.0)
    acc = jnp.dot(pm.astype(jnp.bfloat16), g_bf,
                  preferred_element_type=jnp.float32)
    l = acc[:, dh:dh + 1]
    linv = jnp.where(l > 0, 1.0 / l, 0.0)
    oneh = (nt == jax.lax.broadcasted_iota(jnp.int32, (bm, ntyp), 1)
            ).astype(jnp.float32)
    tbias = jnp.dot(oneh, tb_ref[...], preferred_element_type=jnp.float32)
    return jnp.maximum(acc[:, :dh] * linv + tbias, 0.0)


def _accum_rows(i, idx_ref, hout, zrow_ref, rsum_ref, bm):
    rows_g = jax.lax.broadcasted_iota(jnp.int32, (bm, 1), 0) + i * bm
    zc = jnp.sum(jnp.where(rows_g == idx_ref[0, 0, 0], hout, 0.0),
                 axis=0, keepdims=True)
    rs = jnp.sum(hout, axis=0, keepdims=True)

    @pl.when(i == 0)
    def _():
        zrow_ref[0] = zc
        rsum_ref[0] = rs

    @pl.when(i > 0)
    def _():
        zrow_ref[0] += zc
        rsum_ref[0] += rs


def _store_gext(g, gb_scr, dh):
    n = g.shape[0]
    gb_scr[:, :dh] = g.astype(jnp.bfloat16)
    ones_col = (jax.lax.broadcasted_iota(jnp.int32, (n, 128), 1) == 0)
    gb_scr[:, dh:] = ones_col.astype(jnp.bfloat16)


def _layer1_body(h_ref, w_ref, asrc_ref, adst_ref, z_ref, idx_ref,
                 adj_ref, et_ref, nt_ref, lut_ref, tb_ref,
                 ho_ref, v_ref, zrow_ref, rsum_ref,
                 gb_scr, ssrc_scr, sdst_scr, *, bm, ntyp, dh):
    i = pl.program_id(1)

    @pl.when(i == 0)
    def _():
        h = h_ref[0]
        n = h.shape[0]
        rows = jax.lax.broadcasted_iota(jnp.int32, (n, 1), 0)
        hz = jnp.where(rows == idx_ref[0, 0, 0], z_ref[0], h)
        g, ssrc, sdst = _scores(hz, w_ref, asrc_ref, adst_ref)
        _store_gext(g, gb_scr, dh)
        ssrc_scr[...] = ssrc
        sdst_scr[...] = sdst

    et = et_ref[0]
    v = jnp.where(adj_ref[0] > 0.5, et + 1, 0)
    v_ref[0] = v.astype(jnp.int8)
    hout = _attn_rows(v, lut_ref, ssrc_scr[pl.ds(i * bm, bm), :],
                      sdst_scr[...], gb_scr[...], nt_ref[0], tb_ref, bm,
                      ntyp, dh)
    ho_ref[0] = hout
    _accum_rows(i, idx_ref, hout, zrow_ref, rsum_ref, bm)


def _layer1(h, W, a_src2, a_dst2, z3, idx3, adj, et, nt3, lut, tb, bm):
    b, n, dh = h.shape
    ntyp = tb.shape[0]
    return pl.pallas_call(
        functools.partial(_layer1_body, bm=bm, ntyp=ntyp, dh=dh),
        grid=(b, n // bm),
        in_specs=[
            pl.BlockSpec((1, n, dh), lambda bb, ii: (bb, 0, 0)),
            pl.BlockSpec((dh, dh), lambda bb, ii: (0, 0)),
            pl.BlockSpec((1, dh), lambda bb, ii: (0, 0)),
            pl.BlockSpec((1, dh), lambda bb, ii: (0, 0)),
            pl.BlockSpec((1, 1, dh), lambda bb, ii: (bb, 0, 0)),
            pl.BlockSpec((1, 1, 1), lambda bb, ii: (bb, 0, 0)),
            pl.BlockSpec((1, bm, n), lambda bb, ii: (bb, ii, 0)),
            pl.BlockSpec((1, bm, n), lambda bb, ii: (bb, ii, 0)),
            pl.BlockSpec((1, bm, 1), lambda bb, ii: (bb, ii, 0)),
            pl.BlockSpec((1, 128), lambda bb, ii: (0, 0)),
            pl.BlockSpec((ntyp, dh), lambda bb, ii: (0, 0)),
        ],
        out_specs=[
            pl.BlockSpec((1, bm, dh), lambda bb, ii: (bb, ii, 0)),
            pl.BlockSpec((1, bm, n), lambda bb, ii: (bb, ii, 0)),
            pl.BlockSpec((1, 1, dh), lambda bb, ii: (bb, 0, 0)),
            pl.BlockSpec((1, 1, dh), lambda bb, ii: (bb, 0, 0)),
        ],
        out_shape=[
            jax.ShapeDtypeStruct((b, n, dh), jnp.float32),
            jax.ShapeDtypeStruct((b, n, n), jnp.int8),
            jax.ShapeDtypeStruct((b, 1, dh), jnp.float32),
            jax.ShapeDtypeStruct((b, 1, dh), jnp.float32),
        ],
        scratch_shapes=[
            pltpu.VMEM((n, dh + 128), jnp.bfloat16),
            pltpu.VMEM((n, 1), jnp.float32),
            pltpu.VMEM((1, n), jnp.float32),
        ],
        compiler_params=pltpu.CompilerParams(
            dimension_semantics=("parallel", "arbitrary")),
    )(h, W, a_src2, a_dst2, z3, idx3, adj, et, nt3, lut, tb)


def _layer2_body(h_ref, w_ref, asrc_ref, adst_ref, zrs_ref, zrc_ref,
                 fzw_ref, fzb_ref, idx_ref, v_ref, nt_ref, lut_ref, tb_ref,
                 zrow_ref, rsum_ref,
                 gb_scr, ssrc_scr, sdst_scr, *, bm, ntyp, dh):
    i = pl.program_id(1)

    @pl.when(i == 0)
    def _():
        fzw = fzw_ref[...]
        z = (jnp.dot(zrs_ref[0], fzw[:dh], preferred_element_type=jnp.float32)
             + jnp.dot(zrc_ref[0], fzw[dh:],
                       preferred_element_type=jnp.float32)
             + fzb_ref[...])
        h = h_ref[0]
        n = h.shape[0]
        rows = jax.lax.broadcasted_iota(jnp.int32, (n, 1), 0)
        hz = jnp.where(rows == idx_ref[0, 0, 0], z, h)
        g, ssrc, sdst = _scores(hz, w_ref, asrc_ref, adst_ref)
        _store_gext(g, gb_scr, dh)
        ssrc_scr[...] = ssrc
        sdst_scr[...] = sdst

    v = v_ref[0].astype(jnp.int32)
    hout = _attn_rows(v, lut_ref, ssrc_scr[pl.ds(i * bm, bm), :],
                      sdst_scr[...], gb_scr[...], nt_ref[0], tb_ref, bm,
                      ntyp, dh)
    _accum_rows(i, idx_ref, hout, zrow_ref, rsum_ref, bm)


def _layer2(h, W, a_src2, a_dst2, zrow_s, zrow_c, f_z_W, f_z_b2, idx3,
            v, nt3, lut, tb, bm):
    b, n, dh = h.shape
    ntyp = tb.shape[0]
    return pl.pallas_call(
        functools.partial(_layer2_body, bm=bm, ntyp=ntyp, dh=dh),
        grid=(b, n // bm),
        in_specs=[
            pl.BlockSpec((1, n, dh), lambda bb, ii: (bb, 0, 0)),
            pl.BlockSpec((dh, dh), lambda bb, ii: (0, 0)),
            pl.BlockSpec((1, dh), lambda bb, ii: (0, 0)),
            pl.BlockSpec((1, dh), lambda bb, ii: (0, 0)),
            pl.BlockSpec((1, 1, dh), lambda bb, ii: (bb, 0, 0)),
            pl.BlockSpec((1, 1, dh), lambda bb, ii: (bb, 0, 0)),
            pl.BlockSpec((2 * dh, dh), lambda bb, ii: (0, 0)),
            pl.BlockSpec((1, dh), lambda bb, ii: (0, 0)),
            pl.BlockSpec((1, 1, 1), lambda bb, ii: (bb, 0, 0)),
            pl.BlockSpec((1, bm, n), lambda bb, ii: (bb, ii, 0)),
            pl.BlockSpec((1, bm, 1), lambda bb, ii: (bb, ii, 0)),
            pl.BlockSpec((1, 128), lambda bb, ii: (0, 0)),
            pl.BlockSpec((ntyp, dh), lambda bb, ii: (0, 0)),
        ],
        out_specs=[
            pl.BlockSpec((1, 1, dh), lambda bb, ii: (bb, 0, 0)),
            pl.BlockSpec((1, 1, dh), lambda bb, ii: (bb, 0, 0)),
        ],
        out_shape=[
            jax.ShapeDtypeStruct((b, 1, dh), jnp.float32),
            jax.ShapeDtypeStruct((b, 1, dh), jnp.float32),
        ],
        scratch_shapes=[
            pltpu.VMEM((n, dh + 128), jnp.bfloat16),
            pltpu.VMEM((n, 1), jnp.float32),
            pltpu.VMEM((1, n), jnp.float32),
        ],
        compiler_params=pltpu.CompilerParams(
            dimension_semantics=("parallel", "arbitrary")),
    )(h, W, a_src2, a_dst2, zrow_s, zrow_c, f_z_W, f_z_b2, idx3, v, nt3,
      lut, tb)


def _head_body(zrs_ref, rss_ref, zrc_ref, rsc_ref, p_ref, fzw_ref, fzb_ref,
               hw_ref, hb_ref, o_ref, *, dh, n):
    zs = zrs_ref[:, 0, :]
    zc = zrc_ref[:, 0, :]
    pool_s = (rss_ref[:, 0, :] - zs) * (1.0 / (n - 1.0))
    pool_c = (rsc_ref[:, 0, :] - zc) * (1.0 / (n - 1.0))
    fzw = fzw_ref[...]
    z = (jnp.dot(zs, fzw[:dh], preferred_element_type=jnp.float32)
         + jnp.dot(zc, fzw[dh:], preferred_element_type=jnp.float32)
         + fzb_ref[...])
    hw = hw_ref[...]
    logit = (jnp.dot(pool_s, hw[:dh], preferred_element_type=jnp.float32)
             + jnp.dot(pool_c, hw[dh:2 * dh],
                       preferred_element_type=jnp.float32)
             + jnp.dot(p_ref[:, 0, :], hw[2 * dh:3 * dh],
                       preferred_element_type=jnp.float32)
             + jnp.dot(z, hw[3 * dh:], preferred_element_type=jnp.float32)
             + hb_ref[...])
    o_ref[...] = logit


def _head(zrow_s, rsum_s, zrow_c, rsum_c, p3, f_z_W, f_z_b2, head_W,
          head_b2, n):
    b, _, dh = zrow_s.shape
    return pl.pallas_call(
        functools.partial(_head_body, dh=dh, n=n),
        grid=(1,),
        in_specs=[
            pl.BlockSpec((b, 1, dh), lambda _: (0, 0, 0)),
            pl.BlockSpec((b, 1, dh), lambda _: (0, 0, 0)),
            pl.BlockSpec((b, 1, dh), lambda _: (0, 0, 0)),
            pl.BlockSpec((b, 1, dh), lambda _: (0, 0, 0)),
            pl.BlockSpec((b, 1, dh), lambda _: (0, 0, 0)),
            pl.BlockSpec((2 * dh, dh), lambda _: (0, 0)),
            pl.BlockSpec((1, dh), lambda _: (0, 0)),
            pl.BlockSpec((4 * dh, 1), lambda _: (0, 0)),
            pl.BlockSpec((1, 1), lambda _: (0, 0)),
        ],
        out_specs=pl.BlockSpec((b, 1), lambda _: (0, 0)),
        out_shape=jax.ShapeDtypeStruct((b, 1), jnp.float32),
    )(zrow_s, rsum_s, zrow_c, rsum_c, p3, f_z_W, f_z_b2, head_W, head_b2)


def _lut(rel_k):
    r = rel_k.shape[0]
    return jnp.concatenate(
        [jnp.full((1, 1), -1e9, jnp.float32), rel_k.reshape(1, r),
         jnp.zeros((1, 127 - r), jnp.float32)], axis=1)


def kernel(z_features, p_features, scene_node_features, scene_adj,
           scene_edge_types, scene_node_types, scene_node_is_z_index,
           concept_node_features, concept_adj, concept_edge_types,
           concept_node_types, concept_node_is_z_index,
           visual_W, visual_b, kg_W, kg_b, f_z_W, f_z_b, head_W, head_b,
           scene_W, scene_a_src, scene_a_dst, scene_rel, scene_tb,
           concept_W, concept_a_src, concept_a_dst, concept_rel, concept_tb):
    b, dh = z_features.shape
    ns = scene_adj.shape[1]
    nc = concept_adj.shape[1]
    bm = 512

    idx_s3 = scene_node_is_z_index.astype(jnp.int32).reshape(b, 1, 1)
    idx_c3 = concept_node_is_z_index.astype(jnp.int32).reshape(b, 1, 1)
    nt_s3 = scene_node_types.astype(jnp.int32).reshape(b, ns, 1)
    nt_c3 = concept_node_types.astype(jnp.int32).reshape(b, nc, 1)
    et_s = scene_edge_types.astype(jnp.int32)
    et_c = concept_edge_types.astype(jnp.int32)
    z3 = z_features.reshape(b, 1, dh)
    p3 = p_features.reshape(b, 1, dh)
    vb2 = visual_b.reshape(1, dh)
    kb2 = kg_b.reshape(1, dh)
    fzb2 = f_z_b.reshape(1, dh)
    hb2 = head_b.reshape(1, 1)

    lg2 = jnp.float32(1.4426950408889634)

    h_s = _proj(scene_node_features, visual_W, vb2, z3, idx_s3, bm)
    h_c = _proj(concept_node_features, kg_W, kb2, z3, idx_c3, bm)

    h_s, v_s, zrow_s, _ = _layer1(
        h_s, scene_W[0], (scene_a_src[0] * lg2).reshape(1, dh),
        (scene_a_dst[0] * lg2).reshape(1, dh), z3, idx_s3, scene_adj, et_s,
        nt_s3, _lut(scene_rel[0] * lg2), scene_tb[0], bm)
    h_c, v_c, zrow_c, _ = _layer1(
        h_c, concept_W[0], (concept_a_src[0] * lg2).reshape(1, dh),
        (concept_a_dst[0] * lg2).reshape(1, dh), z3, idx_c3, concept_adj,
        et_c, nt_c3, _lut(concept_rel[0] * lg2), concept_tb[0], bm)

    zrow_s2, rsum_s2 = _layer2(
        h_s, scene_W[1], (scene_a_src[1] * lg2).reshape(1, dh),
        (scene_a_dst[1] * lg2).reshape(1, dh), zrow_s, zrow_c, f_z_W, fzb2,
        idx_s3, v_s, nt_s3, _lut(scene_rel[1] * lg2), scene_tb[1], bm)
    zrow_c2, rsum_c2 = _layer2(
        h_c, concept_W[1], (concept_a_src[1] * lg2).reshape(1, dh),
        (concept_a_dst[1] * lg2).reshape(1, dh), zrow_s, zrow_c, f_z_W, fzb2,
        idx_c3, v_c, nt_c3, _lut(concept_rel[1] * lg2), concept_tb[1], bm)

    return _head(zrow_s2, rsum_s2, zrow_c2, rsum_c2, p3, f_z_W, fzb2,
                 head_W, hb2, ns)


# speedup vs baseline: 1.0899x; 1.0338x over previous
import functools

import jax
import jax.numpy as jnp
from jax.experimental import pallas as pl
from jax.experimental.pallas import tpu as pltpu


def _proj_body(x_ref, w_ref, b_ref, z_ref, idx_ref, o_ref, *, bm):
    i = pl.program_id(1)
    xb = x_ref[0]
    h = jnp.dot(xb.astype(jnp.bfloat16), w_ref[...].astype(jnp.bfloat16),
                preferred_element_type=jnp.float32) + b_ref[...]
    rows = jax.lax.broadcasted_iota(jnp.int32, (bm, 1), 0)
    local = idx_ref[0, 0, 0] - i * bm
    o_ref[0] = jnp.where(rows == local, z_ref[0], h)


def _proj(x, W, bias, z3, idx3, bm):
    b, n, din = x.shape
    dh = W.shape[1]
    return pl.pallas_call(
        functools.partial(_proj_body, bm=bm),
        grid=(b, n // bm),
        in_specs=[
            pl.BlockSpec((1, bm, din), lambda bb, ii: (bb, ii, 0)),
            pl.BlockSpec((din, dh), lambda bb, ii: (0, 0)),
            pl.BlockSpec((1, dh), lambda bb, ii: (0, 0)),
            pl.BlockSpec((1, 1, dh), lambda bb, ii: (bb, 0, 0)),
            pl.BlockSpec((1, 1, 1), lambda bb, ii: (bb, 0, 0)),
        ],
        out_specs=pl.BlockSpec((1, bm, dh), lambda bb, ii: (bb, ii, 0)),
        out_shape=jax.ShapeDtypeStruct((b, n, dh), jnp.float32),
        compiler_params=pltpu.CompilerParams(
            dimension_semantics=("parallel", "parallel")),
    )(x, W, bias, z3, idx3)


def _scores(hz, w_ref, asrc_ref, adst_ref):
    g = jnp.dot(hz.astype(jnp.bfloat16), w_ref[...].astype(jnp.bfloat16),
                preferred_element_type=jnp.float32)
    ssrc = jax.lax.dot_general(g, asrc_ref[...], (((1,), (1,)), ((), ())),
                               preferred_element_type=jnp.float32)
    sdst = jax.lax.dot_general(adst_ref[...], g, (((1,), (1,)), ((), ())),
                               preferred_element_type=jnp.float32)
    return g, ssrc, sdst


def _attn_rows(v, lut_ref, ssrc_blk, sdst, g_bf, nt, tb_ref, bm, ntyp, dh):
    tab = jnp.broadcast_to(lut_ref[...], (bm, lut_ref.shape[1]))
    lut = jnp.take_along_axis(tab, v, axis=1)
    e0 = ssrc_blk + sdst + lut
    e = jnp.maximum(e0, 0.2 * e0)
    m = jnp.max(e, axis=1, keepdims=True)
    p = jnp.exp2(e - m)
    l = jnp.sum(p, axis=1, keepdims=True)
    pm = jnp.where(v > 0, p, 0.0)
    acc = jnp.dot(pm.astype(jnp.bfloat16), g_bf,
                  preferred_element_type=jnp.float32)
    oneh = (nt == jax.lax.broadcasted_iota(jnp.int32, (bm, ntyp), 1)
            ).astype(jnp.float32)
    tbias = jnp.dot(oneh, tb_ref[...], preferred_element_type=jnp.float32)
    return jnp.maximum(acc * (1.0 / l) + tbias, 0.0)


def _accum_rows(i, idx_ref, hout, zrow_ref, rsum_ref, bm):
    rows_g = jax.lax.broadcasted_iota(jnp.int32, (bm, 1), 0) + i * bm
    zc = jnp.sum(jnp.where(rows_g == idx_ref[0, 0, 0], hout, 0.0),
                 axis=0, keepdims=True)
    rs = jnp.sum(hout, axis=0, keepdims=True)

    @pl.when(i == 0)
    def _():
        zrow_ref[0] = zc
        rsum_ref[0] = rs

    @pl.when(i > 0)
    def _():
        zrow_ref[0] += zc
        rsum_ref[0] += rs


def _layer1_body(h_ref, w_ref, asrc_ref, adst_ref, z_ref, idx_ref,
                 adj_ref, et_ref, nt_ref, lut_ref, tb_ref,
                 ho_ref, v_ref, zrow_ref, rsum_ref,
                 gb_scr, ssrc_scr, sdst_scr, *, bm, ntyp, dh):
    i = pl.program_id(1)

    @pl.when(i == 0)
    def _():
        h = h_ref[0]
        n = h.shape[0]
        rows = jax.lax.broadcasted_iota(jnp.int32, (n, 1), 0)
        hz = jnp.where(rows == idx_ref[0, 0, 0], z_ref[0], h)
        g, ssrc, sdst = _scores(hz, w_ref, asrc_ref, adst_ref)
        gb_scr[...] = g.astype(jnp.bfloat16)
        ssrc_scr[...] = ssrc
        sdst_scr[...] = sdst

    et = et_ref[0]
    v = jnp.where(adj_ref[0] > 0.5, et + 1, 0)
    v_ref[0] = v.astype(jnp.int8)
    hout = _attn_rows(v, lut_ref, ssrc_scr[pl.ds(i * bm, bm), :],
                      sdst_scr[...], gb_scr[...], nt_ref[0], tb_ref, bm,
                      ntyp, dh)
    ho_ref[0] = hout
    _accum_rows(i, idx_ref, hout, zrow_ref, rsum_ref, bm)


def _layer1(h, W, a_src2, a_dst2, z3, idx3, adj, et, nt3, lut, tb, bm):
    b, n, dh = h.shape
    ntyp = tb.shape[0]
    return pl.pallas_call(
        functools.partial(_layer1_body, bm=bm, ntyp=ntyp, dh=dh),
        grid=(b, n // bm),
        in_specs=[
            pl.BlockSpec((1, n, dh), lambda bb, ii: (bb, 0, 0)),
            pl.BlockSpec((dh, dh), lambda bb, ii: (0, 0)),
            pl.BlockSpec((1, dh), lambda bb, ii: (0, 0)),
            pl.BlockSpec((1, dh), lambda bb, ii: (0, 0)),
            pl.BlockSpec((1, 1, dh), lambda bb, ii: (bb, 0, 0)),
            pl.BlockSpec((1, 1, 1), lambda bb, ii: (bb, 0, 0)),
            pl.BlockSpec((1, bm, n), lambda bb, ii: (bb, ii, 0)),
            pl.BlockSpec((1, bm, n), lambda bb, ii: (bb, ii, 0)),
            pl.BlockSpec((1, bm, 1), lambda bb, ii: (bb, ii, 0)),
            pl.BlockSpec((1, 128), lambda bb, ii: (0, 0)),
            pl.BlockSpec((ntyp, dh), lambda bb, ii: (0, 0)),
        ],
        out_specs=[
            pl.BlockSpec((1, bm, dh), lambda bb, ii: (bb, ii, 0)),
            pl.BlockSpec((1, bm, n), lambda bb, ii: (bb, ii, 0)),
            pl.BlockSpec((1, 1, dh), lambda bb, ii: (bb, 0, 0)),
            pl.BlockSpec((1, 1, dh), lambda bb, ii: (bb, 0, 0)),
        ],
        out_shape=[
            jax.ShapeDtypeStruct((b, n, dh), jnp.float32),
            jax.ShapeDtypeStruct((b, n, n), jnp.int8),
            jax.ShapeDtypeStruct((b, 1, dh), jnp.float32),
            jax.ShapeDtypeStruct((b, 1, dh), jnp.float32),
        ],
        scratch_shapes=[
            pltpu.VMEM((n, dh), jnp.bfloat16),
            pltpu.VMEM((n, 1), jnp.float32),
            pltpu.VMEM((1, n), jnp.float32),
        ],
        compiler_params=pltpu.CompilerParams(
            dimension_semantics=("parallel", "arbitrary")),
    )(h, W, a_src2, a_dst2, z3, idx3, adj, et, nt3, lut, tb)


def _layer2_body(h_ref, w_ref, asrc_ref, adst_ref, zrs_ref, zrc_ref,
                 fzw_ref, fzb_ref, idx_ref, v_ref, nt_ref, lut_ref, tb_ref,
                 zrow_ref, rsum_ref,
                 gb_scr, ssrc_scr, sdst_scr, *, bm, ntyp, dh):
    i = pl.program_id(1)

    @pl.when(i == 0)
    def _():
        fzw = fzw_ref[...]
        z = (jnp.dot(zrs_ref[0], fzw[:dh], preferred_element_type=jnp.float32)
             + jnp.dot(zrc_ref[0], fzw[dh:],
                       preferred_element_type=jnp.float32)
             + fzb_ref[...])
        h = h_ref[0]
        n = h.shape[0]
        rows = jax.lax.broadcasted_iota(jnp.int32, (n, 1), 0)
        hz = jnp.where(rows == idx_ref[0, 0, 0], z, h)
        g, ssrc, sdst = _scores(hz, w_ref, asrc_ref, adst_ref)
        gb_scr[...] = g.astype(jnp.bfloat16)
        ssrc_scr[...] = ssrc
        sdst_scr[...] = sdst

    v = v_ref[0].astype(jnp.int32)
    hout = _attn_rows(v, lut_ref, ssrc_scr[pl.ds(i * bm, bm), :],
                      sdst_scr[...], gb_scr[...], nt_ref[0], tb_ref, bm,
                      ntyp, dh)
    _accum_rows(i, idx_ref, hout, zrow_ref, rsum_ref, bm)


def _layer2(h, W, a_src2, a_dst2, zrow_s, zrow_c, f_z_W, f_z_b2, idx3,
            v, nt3, lut, tb, bm):
    b, n, dh = h.shape
    ntyp = tb.shape[0]
    return pl.pallas_call(
        functools.partial(_layer2_body, bm=bm, ntyp=ntyp, dh=dh),
        grid=(b, n // bm),
        in_specs=[
            pl.BlockSpec((1, n, dh), lambda bb, ii: (bb, 0, 0)),
            pl.BlockSpec((dh, dh), lambda bb, ii: (0, 0)),
            pl.BlockSpec((1, dh), lambda bb, ii: (0, 0)),
            pl.BlockSpec((1, dh), lambda bb, ii: (0, 0)),
            pl.BlockSpec((1, 1, dh), lambda bb, ii: (bb, 0, 0)),
            pl.BlockSpec((1, 1, dh), lambda bb, ii: (bb, 0, 0)),
            pl.BlockSpec((2 * dh, dh), lambda bb, ii: (0, 0)),
            pl.BlockSpec((1, dh), lambda bb, ii: (0, 0)),
            pl.BlockSpec((1, 1, 1), lambda bb, ii: (bb, 0, 0)),
            pl.BlockSpec((1, bm, n), lambda bb, ii: (bb, ii, 0)),
            pl.BlockSpec((1, bm, 1), lambda bb, ii: (bb, ii, 0)),
            pl.BlockSpec((1, 128), lambda bb, ii: (0, 0)),
            pl.BlockSpec((ntyp, dh), lambda bb, ii: (0, 0)),
        ],
        out_specs=[
            pl.BlockSpec((1, 1, dh), lambda bb, ii: (bb, 0, 0)),
            pl.BlockSpec((1, 1, dh), lambda bb, ii: (bb, 0, 0)),
        ],
        out_shape=[
            jax.ShapeDtypeStruct((b, 1, dh), jnp.float32),
            jax.ShapeDtypeStruct((b, 1, dh), jnp.float32),
        ],
        scratch_shapes=[
            pltpu.VMEM((n, dh), jnp.bfloat16),
            pltpu.VMEM((n, 1), jnp.float32),
            pltpu.VMEM((1, n), jnp.float32),
        ],
        compiler_params=pltpu.CompilerParams(
            dimension_semantics=("parallel", "arbitrary")),
    )(h, W, a_src2, a_dst2, zrow_s, zrow_c, f_z_W, f_z_b2, idx3, v, nt3,
      lut, tb)


def _head_body(zrs_ref, rss_ref, zrc_ref, rsc_ref, p_ref, fzw_ref, fzb_ref,
               hw_ref, hb_ref, o_ref, *, dh, n):
    zs = zrs_ref[:, 0, :]
    zc = zrc_ref[:, 0, :]
    pool_s = (rss_ref[:, 0, :] - zs) * (1.0 / (n - 1.0))
    pool_c = (rsc_ref[:, 0, :] - zc) * (1.0 / (n - 1.0))
    fzw = fzw_ref[...]
    z = (jnp.dot(zs, fzw[:dh], preferred_element_type=jnp.float32)
         + jnp.dot(zc, fzw[dh:], preferred_element_type=jnp.float32)
         + fzb_ref[...])
    hw = hw_ref[...]
    logit = (jnp.dot(pool_s, hw[:dh], preferred_element_type=jnp.float32)
             + jnp.dot(pool_c, hw[dh:2 * dh],
                       preferred_element_type=jnp.float32)
             + jnp.dot(p_ref[:, 0, :], hw[2 * dh:3 * dh],
                       preferred_element_type=jnp.float32)
             + jnp.dot(z, hw[3 * dh:], preferred_element_type=jnp.float32)
             + hb_ref[...])
    o_ref[...] = logit


def _head(zrow_s, rsum_s, zrow_c, rsum_c, p3, f_z_W, f_z_b2, head_W,
          head_b2, n):
    b, _, dh = zrow_s.shape
    return pl.pallas_call(
        functools.partial(_head_body, dh=dh, n=n),
        grid=(1,),
        in_specs=[
            pl.BlockSpec((b, 1, dh), lambda _: (0, 0, 0)),
            pl.BlockSpec((b, 1, dh), lambda _: (0, 0, 0)),
            pl.BlockSpec((b, 1, dh), lambda _: (0, 0, 0)),
            pl.BlockSpec((b, 1, dh), lambda _: (0, 0, 0)),
            pl.BlockSpec((b, 1, dh), lambda _: (0, 0, 0)),
            pl.BlockSpec((2 * dh, dh), lambda _: (0, 0)),
            pl.BlockSpec((1, dh), lambda _: (0, 0)),
            pl.BlockSpec((4 * dh, 1), lambda _: (0, 0)),
            pl.BlockSpec((1, 1), lambda _: (0, 0)),
        ],
        out_specs=pl.BlockSpec((b, 1), lambda _: (0, 0)),
        out_shape=jax.ShapeDtypeStruct((b, 1), jnp.float32),
    )(zrow_s, rsum_s, zrow_c, rsum_c, p3, f_z_W, f_z_b2, head_W, head_b2)


def _lut(rel_k):
    r = rel_k.shape[0]
    return jnp.concatenate(
        [jnp.full((1, 1), -1e9, jnp.float32), rel_k.reshape(1, r),
         jnp.zeros((1, 127 - r), jnp.float32)], axis=1)


def kernel(z_features, p_features, scene_node_features, scene_adj,
           scene_edge_types, scene_node_types, scene_node_is_z_index,
           concept_node_features, concept_adj, concept_edge_types,
           concept_node_types, concept_node_is_z_index,
           visual_W, visual_b, kg_W, kg_b, f_z_W, f_z_b, head_W, head_b,
           scene_W, scene_a_src, scene_a_dst, scene_rel, scene_tb,
           concept_W, concept_a_src, concept_a_dst, concept_rel, concept_tb):
    b, dh = z_features.shape
    ns = scene_adj.shape[1]
    nc = concept_adj.shape[1]
    bm = 512

    idx_s3 = scene_node_is_z_index.astype(jnp.int32).reshape(b, 1, 1)
    idx_c3 = concept_node_is_z_index.astype(jnp.int32).reshape(b, 1, 1)
    nt_s3 = scene_node_types.astype(jnp.int32).reshape(b, ns, 1)
    nt_c3 = concept_node_types.astype(jnp.int32).reshape(b, nc, 1)
    et_s = scene_edge_types.astype(jnp.int32)
    et_c = concept_edge_types.astype(jnp.int32)
    z3 = z_features.reshape(b, 1, dh)
    p3 = p_features.reshape(b, 1, dh)
    vb2 = visual_b.reshape(1, dh)
    kb2 = kg_b.reshape(1, dh)
    fzb2 = f_z_b.reshape(1, dh)
    hb2 = head_b.reshape(1, 1)

    lg2 = jnp.float32(1.4426950408889634)

    h_s = _proj(scene_node_features, visual_W, vb2, z3, idx_s3, bm)
    h_c = _proj(concept_node_features, kg_W, kb2, z3, idx_c3, bm)

    h_s, v_s, zrow_s, _ = _layer1(
        h_s, scene_W[0], (scene_a_src[0] * lg2).reshape(1, dh),
        (scene_a_dst[0] * lg2).reshape(1, dh), z3, idx_s3, scene_adj, et_s,
        nt_s3, _lut(scene_rel[0] * lg2), scene_tb[0], bm)
    h_c, v_c, zrow_c, _ = _layer1(
        h_c, concept_W[0], (concept_a_src[0] * lg2).reshape(1, dh),
        (concept_a_dst[0] * lg2).reshape(1, dh), z3, idx_c3, concept_adj,
        et_c, nt_c3, _lut(concept_rel[0] * lg2), concept_tb[0], bm)

    zrow_s2, rsum_s2 = _layer2(
        h_s, scene_W[1], (scene_a_src[1] * lg2).reshape(1, dh),
        (scene_a_dst[1] * lg2).reshape(1, dh), zrow_s, zrow_c, f_z_W, fzb2,
        idx_s3, v_s, nt_s3, _lut(scene_rel[1] * lg2), scene_tb[1], bm)
    zrow_c2, rsum_c2 = _layer2(
        h_c, concept_W[1], (concept_a_src[1] * lg2).reshape(1, dh),
        (concept_a_dst[1] * lg2).reshape(1, dh), zrow_s, zrow_c, f_z_W, fzb2,
        idx_c3, v_c, nt_c3, _lut(concept_rel[1] * lg2), concept_tb[1], bm)

    return _head(zrow_s2, rsum_s2, zrow_c2, rsum_c2, p3, f_z_W, fzb2,
                 head_W, hb2, ns)
